# Initial kernel scaffold; baseline (speedup 1.0000x reference)
import jax
import jax.numpy as jnp
from jax.experimental import pallas as pl


def kernel(x, router_w, w1, w2):
    raise NotImplementedError("write your pallas kernel here")


# speedup vs baseline: 8.7124x; 8.7124x over previous
import jax
import jax.numpy as jnp
from jax.experimental import pallas as pl

NUM_EXPERTS = 8
TOP_K = 2


def _moe_kernel(x_ref, rw_ref, w1_ref, w2_ref, out_ref):
    e = pl.program_id(0)
    x = x_ref[...]
    rw = rw_ref[...]

    logits = jax.lax.dot_general(
        x, rw, (((1,), (1,)), ((), ())),
        preferred_element_type=jnp.float32)
    m = jnp.max(logits, axis=-1, keepdims=True)
    el = jnp.exp(logits - m)
    probs = el / jnp.sum(el, axis=-1, keepdims=True)

    col = jax.lax.broadcasted_iota(jnp.int32, probs.shape, 1)
    big = jnp.int32(NUM_EXPERTS)
    m1 = jnp.max(probs, axis=-1, keepdims=True)
    is1 = probs == m1
    idx1 = jnp.min(jnp.where(is1, col, big), axis=-1, keepdims=True)
    first1 = col == idx1
    probs_wo1 = jnp.where(first1, -1.0, probs)
    m2 = jnp.max(probs_wo1, axis=-1, keepdims=True)
    is2 = probs_wo1 == m2
    idx2 = jnp.min(jnp.where(is2, col, big), axis=-1, keepdims=True)
    first2 = col == idx2
    gates = jnp.where(first1 | first2, probs, 0.0)

    gate_e = jnp.sum(jnp.where(col == e, gates, 0.0), axis=-1,
                     keepdims=True)

    h = jnp.dot(x, w1_ref[0], preferred_element_type=jnp.float32)
    h = jnp.maximum(h, 0.0)
    y = jnp.dot(h, w2_ref[0], preferred_element_type=jnp.float32)
    contrib = gate_e * y

    @pl.when(e == 0)
    def _():
        out_ref[...] = contrib

    @pl.when(e != 0)
    def _():
        out_ref[...] += contrib


def kernel(x, router_w, w1, w2):
    B, T, C = x.shape
    N = B * T
    E, _, F = w1.shape
    x_flat = x.reshape(N, C)

    out = pl.pallas_call(
        _moe_kernel,
        grid=(E,),
        in_specs=[
            pl.BlockSpec((N, C), lambda e: (0, 0)),
            pl.BlockSpec((E, C), lambda e: (0, 0)),
            pl.BlockSpec((1, C, F), lambda e: (e, 0, 0)),
            pl.BlockSpec((1, F, C), lambda e: (e, 0, 0)),
        ],
        out_specs=pl.BlockSpec((N, C), lambda e: (0, 0)),
        out_shape=jax.ShapeDtypeStruct((N, C), jnp.float32),
    )(x_flat, router_w, w1, w2)
    return out.reshape(B, T, C)
